# baseline (speedup 1.0000x reference)
import functools

import jax
import jax.numpy as jnp
from jax import lax
from jax.experimental import pallas as pl
from jax.experimental.pallas import tpu as pltpu
from jax.experimental.pallas import tpu_sc as plsc

N_NODES = 10000
N_EDGES = 320000
NC = 2
NS = 16
NW = NC * NS
EPW = N_EDGES // NW
CH_H = 80
NFULL_H = EPW // CH_H
TAIL_H = EPW - NFULL_H * CH_H
CH_S = 80
NFULL_S = EPW // CH_S
TAIL_S = EPW - NFULL_S * CH_S
N_PAD = 10112
RPS = N_PAD // NS


def _sc_mesh():
    return plsc.VectorSubcoreMesh(core_axis_name="c", subcore_axis_name="s")


def _sc_degree_histogram(dst, zeros16, ones16):
    @functools.partial(
        pl.kernel,
        out_type=jax.ShapeDtypeStruct((NC, N_PAD, 16), jnp.float32),
        mesh=_sc_mesh(),
        scratch_types=[
            pltpu.VMEM((CH_H,), jnp.int32),
            pltpu.VMEM((CH_H,), jnp.int32),
            pltpu.VMEM((max(TAIL_H, 8),), jnp.int32),
            pltpu.VMEM((CH_H, 16), jnp.float32),
            pltpu.VMEM((max(TAIL_H, 8), 16), jnp.float32),
            pltpu.VMEM_SHARED((N_PAD, 16), jnp.float32),
            pltpu.SemaphoreType.DMA,
            pltpu.SemaphoreType.DMA,
        ],
    )
    def hist(dst_hbm, z_hbm, one_hbm, out_hbm, ia, ib, it, ones_v, ones_t,
             acc_sh, sa, sb):
        c = lax.axis_index("c")
        s = lax.axis_index("s")
        w = c * NS + s
        base = w * EPW
        row0 = s * RPS
        pltpu.sync_copy(one_hbm, ones_v)
        pltpu.sync_copy(one_hbm.at[pl.ds(0, max(TAIL_H, 8))], ones_t)
        pltpu.sync_copy(z_hbm.at[pl.ds(row0, RPS)], acc_sh.at[pl.ds(row0, RPS)])
        plsc.subcore_barrier()

        def start(k, buf, sem):
            pltpu.async_copy(dst_hbm.at[pl.ds(base + k * CH_H, CH_H)], buf, sem)

        def finish(k, buf, sem):
            pltpu.make_async_copy(dst_hbm.at[pl.ds(base + k * CH_H, CH_H)], buf, sem).wait()
            pltpu.sync_copy(ones_v, acc_sh.at[buf], add=True)

        start(0, ia, sa)

        @pl.loop(0, NFULL_H - 1, step=2)
        def _(k):
            start(k + 1, ib, sb)
            finish(k, ia, sa)
            start(k + 2, ia, sa)
            finish(k + 1, ib, sb)

        finish(NFULL_H - 1, ia, sa)
        if TAIL_H:
            pltpu.sync_copy(dst_hbm.at[pl.ds(base + NFULL_H * CH_H, TAIL_H)], it)
            pltpu.sync_copy(ones_t, acc_sh.at[it], add=True)

        plsc.subcore_barrier()
        pltpu.sync_copy(acc_sh.at[pl.ds(row0, RPS)],
                        out_hbm.at[c, pl.ds(row0, RPS)])

    return hist(dst, zeros16, ones16)


def _sc_scatter_rows(g, src, dst, zeros):
    d = g.shape[1]

    @functools.partial(
        pl.kernel,
        out_type=jax.ShapeDtypeStruct((NC, N_PAD, d), jnp.float32),
        mesh=_sc_mesh(),
        compiler_params=pltpu.CompilerParams(use_tc_tiling_on_sc=False) if d == 64 else None,
        scratch_types=[
            pltpu.VMEM((CH_S,), jnp.int32),
            pltpu.VMEM((CH_S,), jnp.int32),
            pltpu.VMEM((CH_S,), jnp.int32),
            pltpu.VMEM((CH_S,), jnp.int32),
            pltpu.VMEM((CH_S,), jnp.int32),
            pltpu.VMEM((CH_S,), jnp.int32),
            pltpu.VMEM((CH_S, d), jnp.float32),
            pltpu.VMEM((CH_S, d), jnp.float32),
            pltpu.VMEM((CH_S, d), jnp.float32),
            pltpu.VMEM_SHARED((N_PAD, d), jnp.float32),
            pltpu.SemaphoreType.DMA,
            pltpu.SemaphoreType.DMA,
            pltpu.SemaphoreType.DMA,
            pltpu.SemaphoreType.DMA,
            pltpu.SemaphoreType.DMA,
            pltpu.SemaphoreType.DMA,
            pltpu.SemaphoreType.DMA,
            pltpu.SemaphoreType.DMA,
            pltpu.SemaphoreType.DMA,
        ],
    )
    def scat(g_hbm, src_hbm, dst_hbm, z_hbm, out_hbm,
             sv0, sv1, sv2, dv0, dv1, dv2, r0, r1, r2, acc_sh,
             i0, i1, i2, gs0, gs1, gs2, c0, c1, c2):
        c = lax.axis_index("c")
        s = lax.axis_index("s")
        w = c * NS + s
        base = w * EPW
        row0 = s * RPS
        pltpu.sync_copy(z_hbm.at[pl.ds(row0, RPS)], acc_sh.at[pl.ds(row0, RPS)])
        plsc.subcore_barrier()

        slots = [(sv0, dv0, r0, i0, gs0, c0),
                 (sv1, dv1, r1, i1, gs1, c1),
                 (sv2, dv2, r2, i2, gs2, c2)]

        def start_idx(k, sl):
            sv, dv, _, isem, _, _ = sl
            pltpu.async_copy(src_hbm.at[pl.ds(base + k * CH_S, CH_S)], sv, isem)
            pltpu.async_copy(dst_hbm.at[pl.ds(base + k * CH_S, CH_S)], dv, isem)

        def start_gather(k, sl):
            sv, dv, rows, isem, gsem, _ = sl
            pltpu.make_async_copy(src_hbm.at[pl.ds(base + k * CH_S, CH_S)], sv, isem).wait()
            pltpu.make_async_copy(dst_hbm.at[pl.ds(base + k * CH_S, CH_S)], dv, isem).wait()
            pltpu.async_copy(g_hbm.at[sv], rows, gsem)

        def start_scat(k, sl):
            sv, dv, rows, _, gsem, csem = sl
            pltpu.make_async_copy(g_hbm.at[sv], rows, gsem).wait()
            pltpu.async_copy(rows, acc_sh.at[dv], csem, add=True)

        def wait_scat(sl):
            _, dv, rows, _, _, csem = sl
            pltpu.make_async_copy(rows, acc_sh.at[dv], csem).wait()

        start_idx(0, slots[0])
        start_idx(1, slots[1])
        start_gather(0, slots[0])
        start_scat(0, slots[0])
        start_idx(2, slots[2])
        start_gather(1, slots[1])

        @pl.loop(1, 121, step=3)
        def _(k):
            for jj in range(3):
                x = slots[(1 + jj) % 3]
                y = slots[(2 + jj) % 3]
                z = slots[jj % 3]
                start_scat(k + jj, x)
                wait_scat(z)
                start_idx(k + jj + 2, z)
                start_gather(k + jj + 1, y)

        start_scat(121, slots[1])
        wait_scat(slots[0])
        start_idx(123, slots[0])
        start_gather(122, slots[2])
        start_scat(122, slots[2])
        wait_scat(slots[1])
        start_idx(124, slots[1])
        start_gather(123, slots[0])
        start_scat(123, slots[0])
        wait_scat(slots[2])
        start_gather(124, slots[1])
        start_scat(124, slots[1])
        wait_scat(slots[0])
        wait_scat(slots[1])

        plsc.subcore_barrier()
        pltpu.sync_copy(acc_sh.at[pl.ds(row0, RPS)],
                        out_hbm.at[c, pl.ds(row0, RPS)])

    return scat(g, src, dst, zeros)


def _dinv(a, b):
    return 1.0 / jnp.sqrt(a[:, :1] + b[:, :1] + 1.0)


def _tc_matmul(a, w):
    m, n = a.shape[0], w.shape[1]

    def body(a_ref, w_ref, o_ref):
        o_ref[...] = lax.dot_general(
            a_ref[...], w_ref[...], (((1,), (0,)), ((), ())),
            preferred_element_type=jnp.float32,
            precision=lax.Precision.HIGHEST)

    return pl.pallas_call(
        body, out_shape=jax.ShapeDtypeStruct((m, n), jnp.float32))(a, w)


def _tc_scale(h, c0, c1):
    def body(h_ref, a_ref, b_ref, o_ref):
        o_ref[...] = h_ref[...] * _dinv(a_ref[...], b_ref[...])

    return pl.pallas_call(
        body, out_shape=jax.ShapeDtypeStruct(h.shape, jnp.float32))(h, c0, c1)


def _tc_mid(p0, p1, g1, c0, c1, w2, b1):
    m, n = g1.shape[0], w2.shape[1]

    def body(p0_ref, p1_ref, g1_ref, a_ref, b_ref, w_ref, bias_ref, o_ref):
        dinv = _dinv(a_ref[...], b_ref[...])
        h = (p0_ref[...] + p1_ref[...] + g1_ref[...]) * dinv + bias_ref[...]
        h = jnp.maximum(h, 0.0)
        o_ref[...] = lax.dot_general(
            h, w_ref[...], (((1,), (0,)), ((), ())),
            preferred_element_type=jnp.float32,
            precision=lax.Precision.HIGHEST) * dinv

    return pl.pallas_call(
        body, out_shape=jax.ShapeDtypeStruct((m, n), jnp.float32))(
            p0, p1, g1, c0, c1, w2, b1)


def _tc_final(q0, q1, g2, c0, c1, b2):
    def body(q0_ref, q1_ref, g2_ref, a_ref, b_ref, bias_ref, o_ref):
        dinv = _dinv(a_ref[...], b_ref[...])
        z = (q0_ref[...] + q1_ref[...] + g2_ref[...]) * dinv + bias_ref[...]
        zm = z - jnp.max(z, axis=1, keepdims=True)
        o_ref[...] = zm - jnp.log(jnp.sum(jnp.exp(zm), axis=1, keepdims=True))

    return pl.pallas_call(
        body, out_shape=jax.ShapeDtypeStruct(g2.shape, jnp.float32))(
            q0, q1, g2, c0, c1, b2)


def kernel(x, edge_index, W1, b1, W2, b2):
    src = edge_index[0]
    dst = edge_index[1]
    zeros16 = jnp.zeros((N_PAD, 16), jnp.float32)
    zeros128 = jnp.zeros((N_PAD, 128), jnp.float32)
    zeros64 = jnp.zeros((N_PAD, 64), jnp.float32)
    ones16 = jnp.ones((CH_H, 16), jnp.float32)

    hist = _sc_degree_histogram(dst, zeros16, ones16)
    c0, c1 = hist[0, :N_NODES], hist[1, :N_NODES]
    hraw = _tc_matmul(x, W1)
    g1 = _tc_scale(hraw, c0, c1)
    p = _sc_scatter_rows(g1, src, dst, zeros128)
    g2 = _tc_mid(p[0, :N_NODES], p[1, :N_NODES], g1, c0, c1, W2, b1.reshape(1, -1))
    q = _sc_scatter_rows(g2, src, dst, zeros64)
    return _tc_final(q[0, :N_NODES], q[1, :N_NODES], g2,
                     c0, c1, b2.reshape(1, -1))


# speedup vs baseline: 1.0829x; 1.0829x over previous
import dataclasses
import functools

import jax
import jax.numpy as jnp
from jax import lax
from jax.experimental import pallas as pl
from jax.experimental.pallas import tpu as pltpu
from jax.experimental.pallas import tpu_sc as plsc

N_NODES = 10000
N_EDGES = 320000
NC = 2
NS = 16
NW = NC * NS
EPW = N_EDGES // NW
CH_H = 80
NFULL_H = EPW // CH_H
TAIL_H = EPW - NFULL_H * CH_H
CH_S = 80
NFULL_S = EPW // CH_S
TAIL_S = EPW - NFULL_S * CH_S
N_PAD = 10112
RPS = N_PAD // NS


def _sc_mesh():
    return plsc.VectorSubcoreMesh(core_axis_name="c", subcore_axis_name="s")


def _sc_degree_histogram(dst):
    cp = pltpu.CompilerParams()
    if "needs_layout_passes" in pltpu.CompilerParams.__dataclass_fields__:
        cp = dataclasses.replace(cp, needs_layout_passes=False)

    @functools.partial(
        pl.kernel,
        out_type=jax.ShapeDtypeStruct((NW * N_NODES,), jnp.float32),
        mesh=_sc_mesh(),
        compiler_params=cp,
        scratch_types=[
            pltpu.VMEM((CH_H,), jnp.int32),
            pltpu.VMEM((CH_H,), jnp.int32),
            pltpu.VMEM((N_NODES,), jnp.float32),
            pltpu.SemaphoreType.DMA,
            pltpu.SemaphoreType.DMA,
        ],
    )
    def hist(dst_hbm, out_hbm, ia, ib, cnt, sa, sb):
        c = lax.axis_index("c")
        s = lax.axis_index("s")
        w = c * NS + s
        base = w * EPW
        zero16 = jnp.zeros((16,), jnp.float32)
        one16 = jnp.ones((16,), jnp.float32)

        @pl.loop(0, N_NODES // 16)
        def _(i):
            cnt[pl.ds(i * 16, 16)] = zero16

        def start(k, buf, sem):
            pltpu.async_copy(dst_hbm.at[pl.ds(base + k * CH_H, CH_H)], buf, sem)

        def finish(k, buf, sem):
            pltpu.make_async_copy(dst_hbm.at[pl.ds(base + k * CH_H, CH_H)], buf, sem).wait()
            for j in range(CH_H // 16):
                plsc.addupdate_scatter(cnt, [buf[pl.ds(j * 16, 16)]], one16)

        start(0, ia, sa)

        @pl.loop(0, NFULL_H - 1, step=2)
        def _(k):
            start(k + 1, ib, sb)
            finish(k, ia, sa)
            start(k + 2, ia, sa)
            finish(k + 1, ib, sb)

        finish(NFULL_H - 1, ia, sa)

        pltpu.sync_copy(cnt, out_hbm.at[pl.ds(w * N_NODES, N_NODES)])

    return hist(dst)


def _sc_scatter_rows(g, src, dst, zeros):
    d = g.shape[1]

    @functools.partial(
        pl.kernel,
        out_type=jax.ShapeDtypeStruct((NC, N_PAD, d), jnp.float32),
        mesh=_sc_mesh(),
        compiler_params=pltpu.CompilerParams(use_tc_tiling_on_sc=False) if d == 64 else None,
        scratch_types=[
            pltpu.VMEM((CH_S,), jnp.int32),
            pltpu.VMEM((CH_S,), jnp.int32),
            pltpu.VMEM((CH_S,), jnp.int32),
            pltpu.VMEM((CH_S,), jnp.int32),
            pltpu.VMEM((CH_S,), jnp.int32),
            pltpu.VMEM((CH_S,), jnp.int32),
            pltpu.VMEM((CH_S, d), jnp.float32),
            pltpu.VMEM((CH_S, d), jnp.float32),
            pltpu.VMEM((CH_S, d), jnp.float32),
            pltpu.VMEM_SHARED((N_PAD, d), jnp.float32),
            pltpu.SemaphoreType.DMA,
            pltpu.SemaphoreType.DMA,
            pltpu.SemaphoreType.DMA,
            pltpu.SemaphoreType.DMA,
            pltpu.SemaphoreType.DMA,
            pltpu.SemaphoreType.DMA,
            pltpu.SemaphoreType.DMA,
            pltpu.SemaphoreType.DMA,
            pltpu.SemaphoreType.DMA,
        ],
    )
    def scat(g_hbm, src_hbm, dst_hbm, z_hbm, out_hbm,
             sv0, sv1, sv2, dv0, dv1, dv2, r0, r1, r2, acc_sh,
             i0, i1, i2, gs0, gs1, gs2, c0, c1, c2):
        c = lax.axis_index("c")
        s = lax.axis_index("s")
        w = c * NS + s
        base = w * EPW
        row0 = s * RPS
        pltpu.sync_copy(z_hbm.at[pl.ds(row0, RPS)], acc_sh.at[pl.ds(row0, RPS)])
        plsc.subcore_barrier()

        slots = [(sv0, dv0, r0, i0, gs0, c0),
                 (sv1, dv1, r1, i1, gs1, c1),
                 (sv2, dv2, r2, i2, gs2, c2)]

        def start_idx(k, sl):
            sv, dv, _, isem, _, _ = sl
            pltpu.async_copy(src_hbm.at[pl.ds(base + k * CH_S, CH_S)], sv, isem)
            pltpu.async_copy(dst_hbm.at[pl.ds(base + k * CH_S, CH_S)], dv, isem)

        def start_gather(k, sl):
            sv, dv, rows, isem, gsem, _ = sl
            pltpu.make_async_copy(src_hbm.at[pl.ds(base + k * CH_S, CH_S)], sv, isem).wait()
            pltpu.make_async_copy(dst_hbm.at[pl.ds(base + k * CH_S, CH_S)], dv, isem).wait()
            pltpu.async_copy(g_hbm.at[sv], rows, gsem)

        def start_scat(k, sl):
            sv, dv, rows, _, gsem, csem = sl
            pltpu.make_async_copy(g_hbm.at[sv], rows, gsem).wait()
            pltpu.async_copy(rows, acc_sh.at[dv], csem, add=True)

        def wait_scat(sl):
            _, dv, rows, _, _, csem = sl
            pltpu.make_async_copy(rows, acc_sh.at[dv], csem).wait()

        start_idx(0, slots[0])
        start_idx(1, slots[1])
        start_gather(0, slots[0])
        start_scat(0, slots[0])
        start_idx(2, slots[2])
        start_gather(1, slots[1])

        @pl.loop(1, 121, step=3)
        def _(k):
            for jj in range(3):
                x = slots[(1 + jj) % 3]
                y = slots[(2 + jj) % 3]
                z = slots[jj % 3]
                start_scat(k + jj, x)
                wait_scat(z)
                start_idx(k + jj + 2, z)
                start_gather(k + jj + 1, y)

        start_scat(121, slots[1])
        wait_scat(slots[0])
        start_idx(123, slots[0])
        start_gather(122, slots[2])
        start_scat(122, slots[2])
        wait_scat(slots[1])
        start_idx(124, slots[1])
        start_gather(123, slots[0])
        start_scat(123, slots[0])
        wait_scat(slots[2])
        start_gather(124, slots[1])
        start_scat(124, slots[1])
        wait_scat(slots[0])
        wait_scat(slots[1])

        plsc.subcore_barrier()
        pltpu.sync_copy(acc_sh.at[pl.ds(row0, RPS)],
                        out_hbm.at[c, pl.ds(row0, RPS)])

    return scat(g, src, dst, zeros)


def _tc_matmul(a, w):
    m, n = a.shape[0], w.shape[1]

    def body(a_ref, w_ref, o_ref):
        o_ref[...] = lax.dot_general(
            a_ref[...], w_ref[...], (((1,), (0,)), ((), ())),
            preferred_element_type=jnp.float32,
            precision=lax.Precision.HIGHEST)

    return pl.pallas_call(
        body, out_shape=jax.ShapeDtypeStruct((m, n), jnp.float32))(a, w)


def _tc_scale(h, parts):
    def body(h_ref, p_ref, o_ref, dinv_ref):
        deg = jnp.sum(p_ref[...], axis=0)[:, None] + 1.0
        dinv = 1.0 / jnp.sqrt(deg)
        dinv_ref[...] = dinv
        o_ref[...] = h_ref[...] * dinv

    return pl.pallas_call(
        body, out_shape=[jax.ShapeDtypeStruct(h.shape, jnp.float32),
                         jax.ShapeDtypeStruct((h.shape[0], 1), jnp.float32)])(h, parts)


def _tc_mid(p0, p1, g1, dinv, w2, b1):
    m, n = g1.shape[0], w2.shape[1]

    def body(p0_ref, p1_ref, g1_ref, dinv_ref, w_ref, bias_ref, o_ref):
        dinv = dinv_ref[...]
        h = (p0_ref[...] + p1_ref[...] + g1_ref[...]) * dinv + bias_ref[...]
        h = jnp.maximum(h, 0.0)
        o_ref[...] = lax.dot_general(
            h, w_ref[...], (((1,), (0,)), ((), ())),
            preferred_element_type=jnp.float32,
            precision=lax.Precision.HIGHEST) * dinv

    return pl.pallas_call(
        body, out_shape=jax.ShapeDtypeStruct((m, n), jnp.float32))(
            p0, p1, g1, dinv, w2, b1)


def _tc_final(q0, q1, g2, dinv, b2):
    def body(q0_ref, q1_ref, g2_ref, dinv_ref, bias_ref, o_ref):
        z = (q0_ref[...] + q1_ref[...] + g2_ref[...]) * dinv_ref[...] + bias_ref[...]
        zm = z - jnp.max(z, axis=1, keepdims=True)
        o_ref[...] = zm - jnp.log(jnp.sum(jnp.exp(zm), axis=1, keepdims=True))

    return pl.pallas_call(
        body, out_shape=jax.ShapeDtypeStruct(g2.shape, jnp.float32))(
            q0, q1, g2, dinv, b2)


def kernel(x, edge_index, W1, b1, W2, b2):
    src = edge_index[0]
    dst = edge_index[1]
    zeros128 = jnp.zeros((N_PAD, 128), jnp.float32)
    zeros64 = jnp.zeros((N_PAD, 64), jnp.float32)

    parts = _sc_degree_histogram(dst).reshape(NW, N_NODES)
    hraw = _tc_matmul(x, W1)
    g1, dinv = _tc_scale(hraw, parts)
    p = _sc_scatter_rows(g1, src, dst, zeros128)
    g2 = _tc_mid(p[0, :N_NODES], p[1, :N_NODES], g1, dinv, W2, b1.reshape(1, -1))
    q = _sc_scatter_rows(g2, src, dst, zeros64)
    return _tc_final(q[0, :N_NODES], q[1, :N_NODES], g2, dinv, b2.reshape(1, -1))


# speedup vs baseline: 1.1325x; 1.0458x over previous
import dataclasses
import functools

import jax
import jax.numpy as jnp
from jax import lax
from jax.experimental import pallas as pl
from jax.experimental.pallas import tpu as pltpu
from jax.experimental.pallas import tpu_sc as plsc

N_NODES = 10000
N_EDGES = 320000
NC = 2
NS = 16
NW = NC * NS
EPW = N_EDGES // NW
CH_H = 80
NFULL_H = EPW // CH_H
TAIL_H = EPW - NFULL_H * CH_H
CH_S = 80
NFULL_S = EPW // CH_S
TAIL_S = EPW - NFULL_S * CH_S
N_PAD = 10112
RPS = N_PAD // NS


def _sc_mesh():
    return plsc.VectorSubcoreMesh(core_axis_name="c", subcore_axis_name="s")


def _sc_degree_histogram(dst):
    cp = pltpu.CompilerParams()
    if "needs_layout_passes" in pltpu.CompilerParams.__dataclass_fields__:
        cp = dataclasses.replace(cp, needs_layout_passes=False)

    @functools.partial(
        pl.kernel,
        out_type=jax.ShapeDtypeStruct((NW * N_NODES,), jnp.float32),
        mesh=_sc_mesh(),
        compiler_params=cp,
        scratch_types=[
            pltpu.VMEM((CH_H,), jnp.int32),
            pltpu.VMEM((CH_H,), jnp.int32),
            pltpu.VMEM((N_NODES,), jnp.float32),
            pltpu.SemaphoreType.DMA,
            pltpu.SemaphoreType.DMA,
        ],
    )
    def hist(dst_hbm, out_hbm, ia, ib, cnt, sa, sb):
        c = lax.axis_index("c")
        s = lax.axis_index("s")
        w = c * NS + s
        base = w * EPW
        zero16 = jnp.zeros((16,), jnp.float32)
        one16 = jnp.ones((16,), jnp.float32)

        @pl.loop(0, N_NODES // 16)
        def _(i):
            cnt[pl.ds(i * 16, 16)] = zero16

        def start(k, buf, sem):
            pltpu.async_copy(dst_hbm.at[pl.ds(base + k * CH_H, CH_H)], buf, sem)

        def finish(k, buf, sem):
            pltpu.make_async_copy(dst_hbm.at[pl.ds(base + k * CH_H, CH_H)], buf, sem).wait()
            for j in range(CH_H // 16):
                plsc.addupdate_scatter(cnt, [buf[pl.ds(j * 16, 16)]], one16)

        start(0, ia, sa)

        @pl.loop(0, NFULL_H - 1, step=2)
        def _(k):
            start(k + 1, ib, sb)
            finish(k, ia, sa)
            start(k + 2, ia, sa)
            finish(k + 1, ib, sb)

        finish(NFULL_H - 1, ia, sa)

        pltpu.sync_copy(cnt, out_hbm.at[pl.ds(w * N_NODES, N_NODES)])

    return hist(dst)


def _sc_scatter_rows(g, src, dst, zeros):
    d = g.shape[1]

    @functools.partial(
        pl.kernel,
        out_type=jax.ShapeDtypeStruct((NC, N_PAD, d), jnp.float32),
        mesh=_sc_mesh(),
        compiler_params=pltpu.CompilerParams(use_tc_tiling_on_sc=False) if d == 64 else None,
        scratch_types=[
            pltpu.VMEM((CH_S,), jnp.int32),
            pltpu.VMEM((CH_S,), jnp.int32),
            pltpu.VMEM((CH_S,), jnp.int32),
            pltpu.VMEM((CH_S,), jnp.int32),
            pltpu.VMEM((CH_S, d), jnp.float32),
            pltpu.VMEM((CH_S, d), jnp.float32),
            pltpu.VMEM_SHARED((N_PAD, d), jnp.float32),
            pltpu.SemaphoreType.DMA,
            pltpu.SemaphoreType.DMA,
            pltpu.SemaphoreType.DMA,
            pltpu.SemaphoreType.DMA,
        ],
    )
    def scat(g_hbm, src_hbm, dst_hbm, z_hbm, out_hbm,
             sa_v, sb_v, da, db, rows_a, rows_b, acc_sh,
             sem_a, sem_b, isem_a, isem_b):
        c = lax.axis_index("c")
        s = lax.axis_index("s")
        w = c * NS + s
        base = w * EPW
        row0 = s * RPS
        pltpu.sync_copy(z_hbm.at[pl.ds(row0, RPS)], acc_sh.at[pl.ds(row0, RPS)])
        plsc.subcore_barrier()

        def start_idx(k, sbuf, dbuf, isem):
            pltpu.async_copy(src_hbm.at[pl.ds(base + k * CH_S, CH_S)], sbuf, isem)
            pltpu.async_copy(dst_hbm.at[pl.ds(base + k * CH_S, CH_S)], dbuf, isem)

        def start_gather(k, sbuf, dbuf, buf, isem, sem):
            pltpu.make_async_copy(src_hbm.at[pl.ds(base + k * CH_S, CH_S)], sbuf, isem).wait()
            pltpu.make_async_copy(dst_hbm.at[pl.ds(base + k * CH_S, CH_S)], dbuf, isem).wait()
            pltpu.async_copy(g_hbm.at[sbuf], buf, sem)

        def finish(k, dbuf, buf, sem):
            pltpu.make_async_copy(g_hbm.at[dbuf], buf, sem).wait()
            pltpu.sync_copy(buf, acc_sh.at[dbuf], add=True)

        start_idx(0, sa_v, da, isem_a)
        start_gather(0, sa_v, da, rows_a, isem_a, sem_a)

        @pl.loop(0, NFULL_S - 1, step=2)
        def _(k):
            start_idx(k + 1, sb_v, db, isem_b)
            start_gather(k + 1, sb_v, db, rows_b, isem_b, sem_b)
            finish(k, da, rows_a, sem_a)
            start_idx(k + 2, sa_v, da, isem_a)
            start_gather(k + 2, sa_v, da, rows_a, isem_a, sem_a)
            finish(k + 1, db, rows_b, sem_b)

        finish(NFULL_S - 1, da, rows_a, sem_a)

        plsc.subcore_barrier()
        pltpu.sync_copy(acc_sh.at[pl.ds(row0, RPS)],
                        out_hbm.at[c, pl.ds(row0, RPS)])

    return scat(g, src, dst, zeros)


def _tc_matmul(a, w):
    m, n = a.shape[0], w.shape[1]

    def body(a_ref, w_ref, o_ref):
        o_ref[...] = lax.dot_general(
            a_ref[...], w_ref[...], (((1,), (0,)), ((), ())),
            preferred_element_type=jnp.float32,
            precision=lax.Precision.HIGHEST)

    return pl.pallas_call(
        body, out_shape=jax.ShapeDtypeStruct((m, n), jnp.float32))(a, w)


def _tc_scale(h, parts):
    def body(h_ref, p_ref, o_ref, dinv_ref):
        deg = jnp.sum(p_ref[...], axis=0)[:, None] + 1.0
        dinv = 1.0 / jnp.sqrt(deg)
        dinv_ref[...] = dinv
        o_ref[...] = h_ref[...] * dinv

    return pl.pallas_call(
        body, out_shape=[jax.ShapeDtypeStruct(h.shape, jnp.float32),
                         jax.ShapeDtypeStruct((h.shape[0], 1), jnp.float32)])(h, parts)


def _tc_mid(p0, p1, g1, dinv, w2, b1):
    m, n = g1.shape[0], w2.shape[1]

    def body(p0_ref, p1_ref, g1_ref, dinv_ref, w_ref, bias_ref, o_ref):
        dinv = dinv_ref[...]
        h = (p0_ref[...] + p1_ref[...] + g1_ref[...]) * dinv + bias_ref[...]
        h = jnp.maximum(h, 0.0)
        o_ref[...] = lax.dot_general(
            h, w_ref[...], (((1,), (0,)), ((), ())),
            preferred_element_type=jnp.float32,
            precision=lax.Precision.HIGHEST) * dinv

    return pl.pallas_call(
        body, out_shape=jax.ShapeDtypeStruct((m, n), jnp.float32))(
            p0, p1, g1, dinv, w2, b1)


def _tc_final(q0, q1, g2, dinv, b2):
    def body(q0_ref, q1_ref, g2_ref, dinv_ref, bias_ref, o_ref):
        z = (q0_ref[...] + q1_ref[...] + g2_ref[...]) * dinv_ref[...] + bias_ref[...]
        zm = z - jnp.max(z, axis=1, keepdims=True)
        o_ref[...] = zm - jnp.log(jnp.sum(jnp.exp(zm), axis=1, keepdims=True))

    return pl.pallas_call(
        body, out_shape=jax.ShapeDtypeStruct(g2.shape, jnp.float32))(
            q0, q1, g2, dinv, b2)


def kernel(x, edge_index, W1, b1, W2, b2):
    src = edge_index[0]
    dst = edge_index[1]
    zeros128 = jnp.zeros((N_PAD, 128), jnp.float32)
    zeros64 = jnp.zeros((N_PAD, 64), jnp.float32)

    parts = _sc_degree_histogram(dst).reshape(NW, N_NODES)
    hraw = _tc_matmul(x, W1)
    g1, dinv = _tc_scale(hraw, parts)
    p = _sc_scatter_rows(g1, src, dst, zeros128)
    g2 = _tc_mid(p[0, :N_NODES], p[1, :N_NODES], g1, dinv, W2, b1.reshape(1, -1))
    q = _sc_scatter_rows(g2, src, dst, zeros64)
    return _tc_final(q[0, :N_NODES], q[1, :N_NODES], g2, dinv, b2.reshape(1, -1))


# speedup vs baseline: 1.2096x; 1.0681x over previous
import dataclasses
import functools

import jax
import jax.numpy as jnp
from jax import lax
from jax.experimental import pallas as pl
from jax.experimental.pallas import tpu as pltpu
from jax.experimental.pallas import tpu_sc as plsc

N_NODES = 10000
N_EDGES = 320000
NC = 2
NS = 16
NW = NC * NS
EPW = N_EDGES // NW
CH_H = 80
NFULL_H = EPW // CH_H
TAIL_H = EPW - NFULL_H * CH_H
CH_S = 80
NFULL_S = EPW // CH_S
TAIL_S = EPW - NFULL_S * CH_S
N_PAD = 10112
RPS = N_PAD // NS


def _sc_mesh():
    return plsc.VectorSubcoreMesh(core_axis_name="c", subcore_axis_name="s")


def _sc_degree_histogram(dst):
    cp = pltpu.CompilerParams()
    if "needs_layout_passes" in pltpu.CompilerParams.__dataclass_fields__:
        cp = dataclasses.replace(cp, needs_layout_passes=False)

    @functools.partial(
        pl.kernel,
        out_type=jax.ShapeDtypeStruct((NW * N_NODES,), jnp.float32),
        mesh=_sc_mesh(),
        compiler_params=cp,
        scratch_types=[
            pltpu.VMEM((EPW,), jnp.int32),
            pltpu.VMEM((N_NODES,), jnp.float32),
            pltpu.SemaphoreType.DMA,
        ],
    )
    def hist(dst_hbm, out_hbm, idx, cnt, sem):
        c = lax.axis_index("c")
        s = lax.axis_index("s")
        w = c * NS + s
        base = w * EPW
        zero16 = jnp.zeros((16,), jnp.float32)
        one16 = jnp.ones((16,), jnp.float32)

        pltpu.async_copy(dst_hbm.at[pl.ds(base, EPW)], idx, sem)

        @pl.loop(0, N_NODES // 16)
        def _(i):
            cnt[pl.ds(i * 16, 16)] = zero16

        pltpu.make_async_copy(dst_hbm.at[pl.ds(base, EPW)], idx, sem).wait()

        @pl.loop(0, EPW // 64)
        def _(e):
            for j in range(4):
                plsc.addupdate_scatter(
                    cnt, [idx[pl.ds(e * 64 + j * 16, 16)]], one16)

        pltpu.sync_copy(cnt, out_hbm.at[pl.ds(w * N_NODES, N_NODES)])

    return hist(dst)


def _sc_scatter_rows(g, src, dst, zeros):
    d = g.shape[1]

    @functools.partial(
        pl.kernel,
        out_type=jax.ShapeDtypeStruct((NC, N_PAD, d), jnp.float32),
        mesh=_sc_mesh(),
        compiler_params=pltpu.CompilerParams(use_tc_tiling_on_sc=False) if d == 64 else None,
        scratch_types=[
            pltpu.VMEM((CH_S,), jnp.int32),
            pltpu.VMEM((CH_S,), jnp.int32),
            pltpu.VMEM((CH_S,), jnp.int32),
            pltpu.VMEM((CH_S,), jnp.int32),
            pltpu.VMEM((CH_S, d), jnp.float32),
            pltpu.VMEM((CH_S, d), jnp.float32),
            pltpu.VMEM_SHARED((N_PAD, d), jnp.float32),
            pltpu.SemaphoreType.DMA,
            pltpu.SemaphoreType.DMA,
            pltpu.SemaphoreType.DMA,
            pltpu.SemaphoreType.DMA,
        ],
    )
    def scat(g_hbm, src_hbm, dst_hbm, z_hbm, out_hbm,
             sa_v, sb_v, da, db, rows_a, rows_b, acc_sh,
             sem_a, sem_b, isem_a, isem_b):
        c = lax.axis_index("c")
        s = lax.axis_index("s")
        w = c * NS + s
        base = w * EPW
        row0 = s * RPS
        pltpu.sync_copy(z_hbm.at[pl.ds(row0, RPS)], acc_sh.at[pl.ds(row0, RPS)])
        plsc.subcore_barrier()

        def start_idx(k, sbuf, dbuf, isem):
            pltpu.async_copy(src_hbm.at[pl.ds(base + k * CH_S, CH_S)], sbuf, isem)
            pltpu.async_copy(dst_hbm.at[pl.ds(base + k * CH_S, CH_S)], dbuf, isem)

        def start_gather(k, sbuf, dbuf, buf, isem, sem):
            pltpu.make_async_copy(src_hbm.at[pl.ds(base + k * CH_S, CH_S)], sbuf, isem).wait()
            pltpu.make_async_copy(dst_hbm.at[pl.ds(base + k * CH_S, CH_S)], dbuf, isem).wait()
            pltpu.async_copy(g_hbm.at[sbuf], buf, sem)

        def finish(k, dbuf, buf, sem):
            pltpu.make_async_copy(g_hbm.at[dbuf], buf, sem).wait()
            pltpu.sync_copy(buf, acc_sh.at[dbuf], add=True)

        start_idx(0, sa_v, da, isem_a)
        start_gather(0, sa_v, da, rows_a, isem_a, sem_a)

        @pl.loop(0, NFULL_S - 1, step=2)
        def _(k):
            start_idx(k + 1, sb_v, db, isem_b)
            start_gather(k + 1, sb_v, db, rows_b, isem_b, sem_b)
            finish(k, da, rows_a, sem_a)
            start_idx(k + 2, sa_v, da, isem_a)
            start_gather(k + 2, sa_v, da, rows_a, isem_a, sem_a)
            finish(k + 1, db, rows_b, sem_b)

        finish(NFULL_S - 1, da, rows_a, sem_a)

        plsc.subcore_barrier()
        pltpu.sync_copy(acc_sh.at[pl.ds(row0, RPS)],
                        out_hbm.at[c, pl.ds(row0, RPS)])

    return scat(g, src, dst, zeros)


def _tc_matmul(a, w):
    m, n = a.shape[0], w.shape[1]

    def body(a_ref, w_ref, o_ref):
        o_ref[...] = lax.dot_general(
            a_ref[...], w_ref[...], (((1,), (0,)), ((), ())),
            preferred_element_type=jnp.float32,
            precision=lax.Precision.HIGHEST)

    return pl.pallas_call(
        body, out_shape=jax.ShapeDtypeStruct((m, n), jnp.float32))(a, w)


def _tc_scale(h, parts):
    def body(h_ref, p_ref, o_ref, dinv_ref):
        deg = jnp.sum(p_ref[...], axis=0)[:, None] + 1.0
        dinv = 1.0 / jnp.sqrt(deg)
        dinv_ref[...] = dinv
        o_ref[...] = h_ref[...] * dinv

    return pl.pallas_call(
        body, out_shape=[jax.ShapeDtypeStruct(h.shape, jnp.float32),
                         jax.ShapeDtypeStruct((h.shape[0], 1), jnp.float32)])(h, parts)


def _tc_mid(p0, p1, g1, dinv, w2, b1):
    m, n = g1.shape[0], w2.shape[1]

    def body(p0_ref, p1_ref, g1_ref, dinv_ref, w_ref, bias_ref, o_ref):
        dinv = dinv_ref[...]
        h = (p0_ref[...] + p1_ref[...] + g1_ref[...]) * dinv + bias_ref[...]
        h = jnp.maximum(h, 0.0)
        o_ref[...] = lax.dot_general(
            h, w_ref[...], (((1,), (0,)), ((), ())),
            preferred_element_type=jnp.float32,
            precision=lax.Precision.HIGHEST) * dinv

    return pl.pallas_call(
        body, out_shape=jax.ShapeDtypeStruct((m, n), jnp.float32))(
            p0, p1, g1, dinv, w2, b1)


def _tc_final(q0, q1, g2, dinv, b2):
    def body(q0_ref, q1_ref, g2_ref, dinv_ref, bias_ref, o_ref):
        z = (q0_ref[...] + q1_ref[...] + g2_ref[...]) * dinv_ref[...] + bias_ref[...]
        zm = z - jnp.max(z, axis=1, keepdims=True)
        o_ref[...] = zm - jnp.log(jnp.sum(jnp.exp(zm), axis=1, keepdims=True))

    return pl.pallas_call(
        body, out_shape=jax.ShapeDtypeStruct(g2.shape, jnp.float32))(
            q0, q1, g2, dinv, b2)


def kernel(x, edge_index, W1, b1, W2, b2):
    src = edge_index[0]
    dst = edge_index[1]
    zeros128 = jnp.zeros((N_PAD, 128), jnp.float32)
    zeros64 = jnp.zeros((N_PAD, 64), jnp.float32)

    parts = _sc_degree_histogram(dst).reshape(NW, N_NODES)
    hraw = _tc_matmul(x, W1)
    g1, dinv = _tc_scale(hraw, parts)
    p = _sc_scatter_rows(g1, src, dst, zeros128)
    g2 = _tc_mid(p[0, :N_NODES], p[1, :N_NODES], g1, dinv, W2, b1.reshape(1, -1))
    q = _sc_scatter_rows(g2, src, dst, zeros64)
    return _tc_final(q[0, :N_NODES], q[1, :N_NODES], g2, dinv, b2.reshape(1, -1))
